# Initial kernel scaffold; baseline (speedup 1.0000x reference)
import jax
import jax.numpy as jnp
from jax.experimental import pallas as pl


def kernel(x, edge_attr, params, edge_index):
    raise NotImplementedError("write your pallas kernel here")


# speedup vs baseline: 1.2525x; 1.2525x over previous
import functools

import jax
import jax.numpy as jnp
from jax import lax
from jax.experimental import pallas as pl
from jax.experimental.pallas import tpu as pltpu
from jax.experimental.pallas import tpu_sc as plsc

F32 = jnp.float32
I32 = jnp.int32

NWORK = 32


def _dotf(a, b):
    return jnp.dot(a, b, preferred_element_type=F32)


def _tc_node_pre(x, Wd, Ws, Gx):
    N, D = x.shape
    BN = 512
    grid = (N + BN - 1) // BN

    def body(x_ref, wd_ref, ws_ref, gx_ref, a_ref, b_ref, g_ref):
        xb = x_ref[...]
        a_ref[...] = _dotf(xb, wd_ref[...])
        b_ref[...] = _dotf(xb, ws_ref[...])
        g_ref[...] = _dotf(xb, gx_ref[...])

    out64 = jax.ShapeDtypeStruct((N, 64), F32)
    wspec = pl.BlockSpec((D, 64), lambda i: (0, 0))
    ospec = pl.BlockSpec((BN, 64), lambda i: (i, 0))
    return pl.pallas_call(
        body,
        grid=(grid,),
        in_specs=[pl.BlockSpec((BN, D), lambda i: (i, 0)), wspec, wspec, wspec],
        out_specs=[ospec, ospec, ospec],
        out_shape=[out64, out64, out64],
    )(x, Wd, Ws, Gx)


def _tc_edge_pre(ea, We1, b1, We2, b2):
    E, DE = ea.shape
    BE = 512
    grid = E // BE

    def body(e_ref, w1_ref, bb1_ref, w2_ref, bb2_ref, c1_ref, c2_ref):
        eb = e_ref[...]
        c1_ref[...] = _dotf(eb, w1_ref[...]) + bb1_ref[...]
        c2_ref[...] = _dotf(eb, w2_ref[...]) + bb2_ref[...]

    out = jax.ShapeDtypeStruct((E, 64), F32)
    wspec = pl.BlockSpec((DE, 64), lambda i: (0, 0))
    bspec = pl.BlockSpec((1, 64), lambda i: (0, 0))
    ospec = pl.BlockSpec((BE, 64), lambda i: (i, 0))
    return pl.pallas_call(
        body,
        grid=(grid,),
        in_specs=[pl.BlockSpec((BE, DE), lambda i: (i, 0)), wspec, bspec,
                  wspec, bspec],
        out_specs=[ospec, ospec],
        out_shape=[out, out],
    )(ea, We1, b1.reshape(1, 64), We2, b2.reshape(1, 64))


SUB = 800
NSUB = 8


def _tc_edge_mlp(Od, Os, C, W2, b2, W3T, b3):
    E = Od.shape[0]
    BE = SUB * NSUB
    grid = E // BE

    def body(a_ref, b_ref, c_ref, w2_ref, bb2_ref, w3t_ref, bb3_ref, o_ref):
        for j in range(NSUB):
            s = slice(j * SUB, (j + 1) * SUB)
            g = a_ref[s, 0:64] + b_ref[s, 64:128] + c_ref[s, :]
            h1 = jnp.maximum(g, 0.0)
            h2 = jnp.maximum(_dotf(h1, w2_ref[...]) + bb2_ref[...], 0.0)
            mt = lax.dot_general(w3t_ref[...], h2, (((1,), (1,)), ((), ())),
                                 preferred_element_type=F32)
            o_ref[:, j, :] = mt + bb3_ref[...]

    gspec = pl.BlockSpec((BE, 128), lambda i: (i, 0))
    espec = pl.BlockSpec((BE, 64), lambda i: (i, 0))
    wspec = pl.BlockSpec((64, 64), lambda i: (0, 0))
    return pl.pallas_call(
        body,
        grid=(grid,),
        in_specs=[gspec, gspec, espec, wspec,
                  pl.BlockSpec((1, 64), lambda i: (0, 0)), wspec,
                  pl.BlockSpec((64, 1), lambda i: (0, 0))],
        out_specs=pl.BlockSpec((64, NSUB, SUB), lambda i: (0, i, 0)),
        out_shape=jax.ShapeDtypeStruct((64, E // SUB, SUB), F32),
    )(Od, Os, C, W2, b2.reshape(1, 64), W3T, b3.reshape(64, 1))


def _agg_dot(agg_ref, gae_ref, gao_ref):
    a0 = agg_ref[:, 0, :]
    a1 = agg_ref[:, 1, :]
    r0 = lax.dot_general(a0, gae_ref[...], (((0,), (0,)), ((), ())),
                         preferred_element_type=F32)
    r1 = lax.dot_general(a1, gao_ref[...], (((0,), (0,)), ((), ())),
                         preferred_element_type=F32)
    return r0 + r1


def _tc_gam1(aggS, xG, GaE, GaO, c1, G2, c2, G3, c3, Wd2, Ws2, Gh2):
    N = xG.shape[0]
    BN = 512
    grid = (N + BN - 1) // BN

    def body(agg_ref, xg_ref, gae_ref, gao_ref, cc1_ref, g2_ref, cc2_ref,
             g3_ref, cc3_ref, wd_ref, ws_ref, gh_ref,
             a2_ref, b2_ref, hg_ref):
        agg = _agg_dot(agg_ref, gae_ref, gao_ref)
        g1 = jnp.maximum(agg + xg_ref[...] + cc1_ref[...], 0.0)
        g2 = jnp.maximum(_dotf(g1, g2_ref[...]) + cc2_ref[...], 0.0)
        h = jnp.maximum(_dotf(g2, g3_ref[...]) + cc3_ref[...], 0.0)
        a2_ref[...] = _dotf(h, wd_ref[...])
        b2_ref[...] = _dotf(h, ws_ref[...])
        hg_ref[...] = _dotf(h, gh_ref[...])

    out64 = jax.ShapeDtypeStruct((N, 64), F32)
    hwspec = pl.BlockSpec((32, 64), lambda i: (0, 0))
    wspec = pl.BlockSpec((64, 64), lambda i: (0, 0))
    bspec = pl.BlockSpec((1, 64), lambda i: (0, 0))
    nspec = pl.BlockSpec((BN, 64), lambda i: (i, 0))
    return pl.pallas_call(
        body,
        grid=(grid,),
        in_specs=[pl.BlockSpec((32, 2, BN), lambda i: (0, 0, i)), nspec,
                  hwspec, hwspec, bspec, wspec, bspec, wspec, bspec,
                  wspec, wspec, wspec],
        out_specs=[nspec, nspec, nspec],
        out_shape=[out64, out64, out64],
    )(aggS, xG, GaE, GaO, c1.reshape(1, 64), G2, c2.reshape(1, 64), G3,
      c3.reshape(1, 64), Wd2, Ws2, Gh2)


def _tc_gam2_head(aggS, hG, GaE, GaO, c1, G2, c2, G3, c3,
                  H1, d1, H2, d2, H3, d3):
    N = hG.shape[0]
    BN = 512
    grid = (N + BN - 1) // BN

    def body(agg_ref, hg_ref, gae_ref, gao_ref, cc1_ref, g2_ref, cc2_ref,
             g3_ref, cc3_ref, h1_ref, dd1_ref, h2_ref, dd2_ref, h3_ref,
             dd3_ref, o_ref):
        agg = _agg_dot(agg_ref, gae_ref, gao_ref)
        g1 = jnp.maximum(agg + hg_ref[...] + cc1_ref[...], 0.0)
        g2 = jnp.maximum(_dotf(g1, g2_ref[...]) + cc2_ref[...], 0.0)
        h = _dotf(g2, g3_ref[...]) + cc3_ref[...]
        y = jnp.maximum(_dotf(h, h1_ref[...]) + dd1_ref[...], 0.0)
        y = jnp.maximum(_dotf(y, h2_ref[...]) + dd2_ref[...], 0.0)
        o_ref[...] = _dotf(y, h3_ref[...]) + dd3_ref[...]

    hwspec = pl.BlockSpec((32, 64), lambda i: (0, 0))
    wspec = pl.BlockSpec((64, 64), lambda i: (0, 0))
    bspec = pl.BlockSpec((1, 64), lambda i: (0, 0))
    nspec = pl.BlockSpec((BN, 64), lambda i: (i, 0))
    return pl.pallas_call(
        body,
        grid=(grid,),
        in_specs=[pl.BlockSpec((32, 2, BN), lambda i: (0, 0, i)), nspec,
                  hwspec, hwspec, bspec, wspec, bspec, wspec, bspec,
                  wspec, bspec, wspec, bspec,
                  pl.BlockSpec((64, 1), lambda i: (0, 0)),
                  pl.BlockSpec((1, 1), lambda i: (0, 0))],
        out_specs=pl.BlockSpec((BN, 1), lambda i: (i, 0)),
        out_shape=jax.ShapeDtypeStruct((N, 1), F32),
    )(aggS, hG, GaE, GaO, c1.reshape(1, 64), G2, c2.reshape(1, 64), G3,
      c3.reshape(1, 64), H1, d1.reshape(1, 64), H2, d2.reshape(1, 64),
      H3, d3.reshape(1, 1))


def _sc_gather(T, dstI, srcI):
    E = dstI.shape[0]
    CH = 128
    nchunk = E // CH
    trips = (nchunk + NWORK - 1) // NWORK
    mesh = plsc.VectorSubcoreMesh(core_axis_name="c", subcore_axis_name="s")

    out = jax.ShapeDtypeStruct((E, 128), F32)

    @functools.partial(
        pl.kernel,
        out_type=[out, out],
        mesh=mesh,
        scratch_types=[
            pltpu.VMEM((CH,), I32), pltpu.VMEM((CH,), I32),
            pltpu.VMEM((CH, 128), F32), pltpu.VMEM((CH, 128), F32),
            pltpu.SemaphoreType.DMA, pltpu.SemaphoreType.DMA,
        ],
        compiler_params=pltpu.CompilerParams(needs_layout_passes=False),
    )
    def k(t_hbm, d_hbm, s_hbm, oa_hbm, ob_hbm,
          idxa, idxb, bufa, bufb, sema, semb):
        wid = lax.axis_index("s") * 2 + lax.axis_index("c")

        def step(i, carry):
            c = wid + i * NWORK

            @pl.when(c < nchunk)
            def _():
                base = c * CH
                pltpu.sync_copy(d_hbm.at[pl.ds(base, CH)], idxa)
                pltpu.sync_copy(s_hbm.at[pl.ds(base, CH)], idxb)
                cpa = pltpu.async_copy(t_hbm.at[idxa], bufa, sema)
                cpb = pltpu.async_copy(t_hbm.at[idxb], bufb, semb)
                cpa.wait()
                cpb.wait()
                pltpu.sync_copy(bufa, oa_hbm.at[pl.ds(base, CH)])
                pltpu.sync_copy(bufb, ob_hbm.at[pl.ds(base, CH)])

            return carry

        lax.fori_loop(0, trips, step, 0)

    return k(T, dstI, srcI)


def _sc_segmax(mT3, dstI, N):
    E = dstI.shape[0]
    CH = SUB * NSUB
    nch = E // CH
    ngrp = CH // 16
    NP = ((N + 15) // 16) * 16
    ninit = NP // 16
    mesh = plsc.VectorSubcoreMesh(core_axis_name="c", subcore_axis_name="s")

    @functools.partial(
        pl.kernel,
        out_type=jax.ShapeDtypeStruct((NWORK, 2, N), F32),
        mesh=mesh,
        scratch_types=[
            pltpu.VMEM((CH,), I32),
            pltpu.VMEM((NSUB, SUB), F32), pltpu.VMEM((NSUB, SUB), F32),
            pltpu.VMEM((NP,), F32), pltpu.VMEM((NP,), F32),
            pltpu.VMEM((2, NP), F32),
            pltpu.VMEM((NP,), I32),
        ],
        compiler_params=pltpu.CompilerParams(needs_layout_passes=False),
    )
    def k(m_hbm, d_hbm, agg_hbm, dbuf, va, vb, acca, accb, accab, sids):
        wid = lax.axis_index("s") * 2 + lax.axis_index("c")
        f0 = wid * 2
        neg = jnp.full((16,), -jnp.inf, F32)

        def init(i, carry):
            acca[pl.ds(i * 16, 16)] = neg
            accb[pl.ds(i * 16, 16)] = neg
            return carry

        lax.fori_loop(0, ninit, init, 0)

        ids = lax.iota(I32, 16)
        gps = SUB // 16

        def chunk(c, carry):
            base = c * CH
            pltpu.sync_copy(d_hbm.at[pl.ds(base, CH)], dbuf)
            pltpu.sync_copy(m_hbm.at[f0, pl.ds(c * NSUB, NSUB)], va)
            pltpu.sync_copy(m_hbm.at[f0 + 1, pl.ds(c * NSUB, NSUB)], vb)

            def group(g, carry2):
                o = g * 16
                d16 = dbuf[pl.ds(o, 16)]
                row = g // gps
                col = (g % gps) * 16
                va16 = va[row, pl.ds(col, 16)]
                vb16 = vb[row, pl.ds(col, 16)]
                plsc.store_scatter(sids, [d16], ids)
                back = plsc.load_gather(sids, [d16])
                allu = jnp.all(back == ids)

                def fast():
                    cura = plsc.load_gather(acca, [d16])
                    plsc.store_scatter(acca, [d16], jnp.maximum(cura, va16))
                    curb = plsc.load_gather(accb, [d16])
                    plsc.store_scatter(accb, [d16], jnp.maximum(curb, vb16))

                def slow():
                    def one(acc, v):
                        def cond(pend):
                            return jnp.any(pend)

                        def body(pend):
                            cur = plsc.load_gather(acc, [d16], mask=pend)
                            new = jnp.maximum(cur, v)
                            plsc.store_scatter(acc, [d16], new, mask=pend)
                            chk = plsc.load_gather(acc, [d16], mask=pend)
                            return pend & (chk < new)

                        lax.while_loop(cond, body, d16 == d16)

                    one(acca, va16)
                    one(accb, vb16)

                lax.cond(allu, fast, slow)
                return carry2

            lax.fori_loop(0, ngrp, group, 0)
            return carry

        lax.fori_loop(0, nch, chunk, 0)

        def fin(i, carry):
            s = pl.ds(i * 16, 16)
            a = acca[s]
            accab[0, s] = jnp.where(a == neg, 0.0, a)
            b = accb[s]
            accab[1, s] = jnp.where(b == neg, 0.0, b)
            return carry

        lax.fori_loop(0, ninit, fin, 0)
        pltpu.sync_copy(accab.at[:, pl.ds(0, N)], agg_hbm.at[wid])

    return k(mT3, dstI)


def kernel(x, edge_attr, params, edge_index):
    D = x.shape[1]

    (W1a, b1a), (W2a, b2a), (W3a, b3a) = params['l1_phi']
    (G1a, c1a), (G2a, c2a), (G3a, c3a) = params['l1_gam']
    (W1b, b1b), (W2b, b2b), (W3b, b3b) = params['l2_phi']
    (G1b, c1b), (G2b, c2b), (G3b, c3b) = params['l2_gam']
    (H1, d1), (H2, d2), (H3, d3) = params['head']

    Wd1, Ws1, We1 = W1a[:D], W1a[D:2 * D], W1a[2 * D:]
    Ga1, Gx1 = G1a[:64], G1a[64:]
    Wd2, Ws2, We2 = W1b[:64], W1b[64:128], W1b[128:]
    Ga2, Gh2 = G1b[:64], G1b[64:]
    Ga1E, Ga1O = Ga1[0::2], Ga1[1::2]
    Ga2E, Ga2O = Ga2[0::2], Ga2[1::2]

    src = edge_index[0].astype(I32)
    dst = edge_index[1].astype(I32)
    N = x.shape[0]

    A1, B1, xG = _tc_node_pre(x, Wd1, Ws1, Gx1)
    C1, C2 = _tc_edge_pre(edge_attr, We1, b1a, We2, b1b)

    Od1, Os1 = _sc_gather(jnp.concatenate([A1, B1], axis=1), dst, src)
    mT1 = _tc_edge_mlp(Od1, Os1, C1, W2a, b2a, W3a.T, b3a)
    aggS1 = _sc_segmax(mT1, dst, N)
    A2, B2, hG = _tc_gam1(aggS1, xG, Ga1E, Ga1O, c1a, G2a, c2a, G3a, c3a,
                          Wd2, Ws2, Gh2)

    Od2, Os2 = _sc_gather(jnp.concatenate([A2, B2], axis=1), dst, src)
    mT2 = _tc_edge_mlp(Od2, Os2, C2, W2b, b2b, W3b.T, b3b)
    aggS2 = _sc_segmax(mT2, dst, N)
    return _tc_gam2_head(aggS2, hG, Ga2E, Ga2O, c1b, G2b, c2b, G3b, c3b,
                         H1, d1, H2, d2, H3, d3)
